# baseline (speedup 1.0000x reference)
import functools

import jax
import jax.numpy as jnp
from jax import lax
from jax.experimental import pallas as pl
from jax.experimental.pallas import tpu as pltpu, tpu_sc as plsc

B, C, H, W = 64, 384, 28, 28
D = H * W
ROWS = B * C
NC, NS, L = 2, 16, 16
NW = NC * NS
BPW = B // NW
RPW = BPW * C
K = 64
NCHUNK = RPW // K
V16 = K // L


def _body(x_hbm, perm_hbm, out_hbm,
          perm_v, idx_v, buf0, buf1, gs0, gs1, ws0, ws1):
    wid = lax.axis_index("s") * NC + lax.axis_index("c")
    out_base = wid * RPW

    pltpu.sync_copy(perm_hbm, perm_v)

    for i in range(NCHUNK):
        row = idx_v.at[i]
        for t in range(V16):
            r0 = i * K + t * L
            off = r0 % C
            batch = wid * BPW + (r0 // C)
            row[pl.ds(t * L, L)] = perm_v[pl.ds(off, L)] + batch * C

    bufs = (buf0, buf1)
    gsems = (gs0, gs1)
    wsems = (ws0, ws1)

    def copy_in(i):
        return pltpu.async_copy(x_hbm.at[idx_v.at[i]], bufs[i % 2],
                                gsems[i % 2])

    def copy_out(i):
        return pltpu.async_copy(bufs[i % 2],
                                out_hbm.at[pl.ds(out_base + i * K, K)],
                                wsems[i % 2])

    g = {}
    w = {}
    g[0] = copy_in(0)
    for i in range(NCHUNK):
        if i + 1 < NCHUNK:
            if i - 1 >= 0:
                w[i - 1].wait()
            g[i + 1] = copy_in(i + 1)
        g[i].wait()
        w[i] = copy_out(i)
    w[NCHUNK - 2].wait()
    w[NCHUNK - 1].wait()


@jax.jit
def _permute(x_rows, perm):
    mesh = plsc.VectorSubcoreMesh(core_axis_name="c", subcore_axis_name="s")
    run = functools.partial(
        pl.kernel,
        mesh=mesh,
        compiler_params=pltpu.CompilerParams(use_tc_tiling_on_sc=False),
        out_type=jax.ShapeDtypeStruct((ROWS, D), jnp.float32),
        scratch_types=[
            pltpu.VMEM((C,), jnp.int32),
            pltpu.VMEM((NCHUNK, K), jnp.int32),
            pltpu.VMEM((K, D), jnp.float32),
            pltpu.VMEM((K, D), jnp.float32),
            pltpu.SemaphoreType.DMA,
            pltpu.SemaphoreType.DMA,
            pltpu.SemaphoreType.DMA,
            pltpu.SemaphoreType.DMA,
        ],
    )(_body)
    return run(x_rows, perm)


def kernel(x, perm):
    y = _permute(x.reshape(ROWS, D), perm).reshape(B, C, H, W)
    logdet = jnp.zeros((B,), dtype=x.dtype)
    return (y, logdet)
